# baseline (speedup 1.0000x reference)
import jax
import jax.numpy as jnp
from jax.experimental import pallas as pl
from jax.experimental.pallas import tpu as pltpu

_BM = 200


def _outer(eps1_ref, eps2_ref, g1_hbm, g2_hbm, x1_ref, x2_ref,
           w10_ref, b10_ref, w11_ref, b11_ref, w12_ref, b12_ref,
           w20_ref, b20_ref, w21_ref, b21_ref, w22_ref, b22_ref,
           wm1_ref, wm2_ref, bm_ref, out_hbm):
    n = x1_ref.shape[0]

    def _step(g1_ref, g2_ref, out_ref):
        h1 = (2.0 + eps1_ref[0, 0]) * jnp.dot(
            g1_ref[...].astype(jnp.bfloat16), x1_ref[...],
            preferred_element_type=jnp.float32)
        h2 = (2.0 + eps2_ref[0, 0]) * jnp.dot(
            g2_ref[...].astype(jnp.bfloat16), x2_ref[...],
            preferred_element_type=jnp.float32)
        h1 = jnp.maximum(jnp.dot(h1, w10_ref[...]) + b10_ref[...], 0.0)
        h2 = jnp.maximum(jnp.dot(h2, w20_ref[...]) + b20_ref[...], 0.0)
        h1 = jnp.maximum(jnp.dot(h1, w11_ref[...]) + b11_ref[...], 0.0)
        h2 = jnp.maximum(jnp.dot(h2, w21_ref[...]) + b21_ref[...], 0.0)
        h1 = jnp.maximum(jnp.dot(h1, w12_ref[...]) + b12_ref[...], 0.0)
        h2 = jnp.maximum(jnp.dot(h2, w22_ref[...]) + b22_ref[...], 0.0)
        out_ref[...] = (jnp.dot(h1, wm1_ref[...]) + jnp.dot(h2, wm2_ref[...])
                        + bm_ref[...])

    def row_block(i):
        return (i, 0)

    g_spec = pl.BlockSpec((_BM, n), row_block,
                          pipeline_mode=pl.Buffered(buffer_count=3))
    pipe = pltpu.emit_pipeline(
        _step,
        grid=(n // _BM,),
        in_specs=[g_spec, g_spec],
        out_specs=[pl.BlockSpec((_BM, out_hbm.shape[1]), row_block)],
    )
    pipe(g1_hbm, g2_hbm, out_hbm)


def kernel(x1, x2, G1, G2, eps1, eps2, W10, b10, W11, b11, W12, b12,
           W20, b20, W21, b21, W22, b22, Wm, bm):
    n, d1 = x1.shape
    d2 = x2.shape[1]
    out = Wm.shape[1]

    eps1_2d = eps1.reshape(1, 1)
    eps2_2d = eps2.reshape(1, 1)
    wm1 = Wm[:out, :]
    wm2 = Wm[out:, :]

    vmem = lambda shape: pl.BlockSpec(shape, memory_space=pltpu.VMEM)
    hbm = pl.BlockSpec(memory_space=pl.ANY)

    return pl.pallas_call(
        _outer,
        in_specs=[
            vmem((1, 1)),
            vmem((1, 1)),
            hbm,
            hbm,
            vmem((n, d1)),
            vmem((n, d2)),
            vmem(W10.shape), vmem((1, out)),
            vmem(W11.shape), vmem((1, out)),
            vmem(W12.shape), vmem((1, out)),
            vmem(W20.shape), vmem((1, out)),
            vmem(W21.shape), vmem((1, out)),
            vmem(W22.shape), vmem((1, out)),
            vmem(wm1.shape), vmem(wm2.shape), vmem((1, out)),
        ],
        out_specs=pl.BlockSpec(memory_space=pl.ANY),
        out_shape=jax.ShapeDtypeStruct((n, out), jnp.float32),
    )(eps1_2d, eps2_2d, G1, G2,
      x1.astype(jnp.bfloat16), x2.astype(jnp.bfloat16),
      W10, b10.reshape(1, -1), W11, b11.reshape(1, -1), W12, b12.reshape(1, -1),
      W20, b20.reshape(1, -1), W21, b21.reshape(1, -1), W22, b22.reshape(1, -1),
      wm1, wm2, bm.reshape(1, -1))
